# baseline (speedup 1.0000x reference)
import functools
import jax
import jax.numpy as jnp
from jax import lax
from jax.experimental import pallas as pl
from jax.experimental.pallas import tpu as pltpu
from jax.experimental.pallas import tpu_sc as plsc

MAXV = 52000.0
MINV = -53000.0
LEV = 1000
NGRAM = 4
D = 4096
C = 23
CP = 24
T = 64
B = 16
PAIRS = B * T
NW = 32
PPW = PAIRS // NW
NH = 2
DH = D // NH
LANES = 16

_mesh = plsc.VectorSubcoreMesh(core_axis_name="c", subcore_axis_name="s")


@functools.partial(
    pl.kernel,
    mesh=_mesh,
    out_type=jax.ShapeDtypeStruct((PAIRS * D,), jnp.float32),
    scratch_types=[
        pltpu.VMEM((CP,), jnp.int32),
        pltpu.VMEM((CP, DH), jnp.float32),
        pltpu.VMEM((C * DH,), jnp.float32),
        pltpu.VMEM((DH,), jnp.float32),
        pltpu.SemaphoreType.DMA,
    ],
)
def _sc_samples(idx_hbm, table2_hbm, ch_hbm, out_hbm,
                idx_v, rows_v, ch_v, acc_v, sem):
    wid = lax.axis_index("s") * 2 + lax.axis_index("c")
    base = wid * PPW
    for h in range(NH):
        pltpu.sync_copy(ch_hbm.at[pl.ds(h * C * DH, C * DH)], ch_v)

        def pair_body(p, _):
            pltpu.sync_copy(
                idx_hbm.at[pl.ds((h * PAIRS + base + p) * CP, CP)], idx_v)
            pltpu.async_copy(table2_hbm.at[idx_v], rows_v, sem).wait()

            def chunk_body(j, _):
                sl = pl.ds(j * LANES, LANES)
                acc = rows_v[0, sl] * ch_v[pl.ds(j * LANES, LANES)]
                for c in range(1, C):
                    acc = acc + rows_v[c, sl] * ch_v[pl.ds(c * DH + j * LANES, LANES)]
                acc_v[sl] = acc
                return 0

            lax.fori_loop(0, DH // LANES, chunk_body, 0)
            pltpu.sync_copy(
                acc_v, out_hbm.at[pl.ds((base + p) * D + h * DH, DH)])
            return 0

        lax.fori_loop(0, PPW, pair_body, 0)


def _ngram_body(s_ref, out_ref):
    samples = s_ref[0]

    def roll1(a):
        return jnp.concatenate([a[:, -1:], a[:, :-1]], axis=1)

    r0 = samples
    r1 = roll1(r0)
    r2 = roll1(r1)
    r3 = roll1(r2)
    w = T - (NGRAM - 1)
    ng = (r3[0:w] * r2[1:w + 1] * r1[2:w + 2] * r0[3:w + 3])
    s = jnp.sum(ng, axis=0, keepdims=True)
    out_ref[...] = jnp.where(s > 0.0, 1.0, -1.0).astype(jnp.float32)[None]


def kernel(input, signals_weight, channels_weight):
    idxf = jnp.round((input - MINV) / (MAXV - MINV) * (LEV - 1))
    idx = jnp.clip(idxf, 0.0, float(LEV - 1)).astype(jnp.int32)
    idx = idx.reshape(PAIRS, C)
    idxp = jnp.pad(idx, ((0, 0), (0, CP - C)), mode="edge")
    idx2 = jnp.concatenate(
        [idxp * NH, idxp * NH + 1], axis=0).reshape(-1)
    table2 = signals_weight.reshape(LEV * NH, DH)
    ch2 = channels_weight.reshape(C, NH, DH).transpose(1, 0, 2).reshape(-1)

    samples = _sc_samples(idx2, table2, ch2)

    out = pl.pallas_call(
        _ngram_body,
        grid=(B,),
        in_specs=[pl.BlockSpec((1, T, D), lambda i: (i, 0, 0))],
        out_specs=pl.BlockSpec((1, 1, D), lambda i: (i, 0, 0)),
        out_shape=jax.ShapeDtypeStruct((B, 1, D), jnp.float32),
    )(samples.reshape(B, T, D))
    return out.reshape(B, D)
